# baseline (speedup 1.0000x reference)
import functools
import math

import jax
import jax.numpy as jnp
from jax import lax
from jax.experimental import pallas as pl
from jax.experimental.pallas import tpu as pltpu
from jax.experimental.pallas import tpu_sc as plsc

DIM = 64
SCALE = math.sqrt(DIM)
CHUNK = 128
LANES = 16


@functools.cache
def _make_sc_lookup(n_rows: int):
    info = plsc.get_sparse_core_info()
    nw = info.num_cores * info.num_subcores
    rows_per_w = n_rows // nw
    assert rows_per_w * nw == n_rows
    nch = rows_per_w // CHUNK
    assert nch * CHUNK == rows_per_w and nch >= 4 and nch % 2 == 0

    mesh = plsc.VectorSubcoreMesh(core_axis_name="c", subcore_axis_name="s")

    @functools.partial(
        pl.kernel,
        out_type=jax.ShapeDtypeStruct((n_rows, DIM), jnp.float32),
        mesh=mesh,
        compiler_params=pltpu.CompilerParams(use_tc_tiling_on_sc=False),
        scratch_types=[
            pltpu.VMEM((nch, CHUNK), jnp.int32),
            pltpu.VMEM((CHUNK, DIM), jnp.float32),
            pltpu.VMEM((CHUNK, DIM), jnp.float32),
            pltpu.VMEM((CHUNK, DIM), jnp.float32),
            pltpu.VMEM((CHUNK, DIM), jnp.float32),
            pltpu.SemaphoreType.DMA,
            pltpu.SemaphoreType.DMA,
            pltpu.SemaphoreType.DMA,
            pltpu.SemaphoreType.DMA,
        ],
    )
    def lookup(idx_hbm, table_hbm, out_hbm, idx_v, g0, g1, o0, o1,
               sg0, sg1, so0, so1):
        gbuf = (g0, g1)
        obuf = (o0, o1)
        gsem = (sg0, sg1)
        osem = (so0, so1)

        wid = lax.axis_index("s") * info.num_cores + lax.axis_index("c")
        base_row = wid * rows_per_w

        pltpu.sync_copy(idx_hbm.at[pl.ds(wid * nch, nch)], idx_v)

        def start_gather(gb, b):
            pltpu.async_copy(table_hbm.at[idx_v.at[gb]], gbuf[b], gsem[b])

        def wait_gather(b):
            pltpu.make_async_copy(
                table_hbm.at[pl.ds(0, CHUNK)], gbuf[b], gsem[b]).wait()

        def scale(b):
            src, dst = gbuf[b], obuf[b]

            @pl.loop(0, CHUNK, unroll=4)
            def _(r):
                for c in range(DIM // LANES):
                    dst[r, pl.ds(c * LANES, LANES)] = (
                        src[r, pl.ds(c * LANES, LANES)] * SCALE)

        def start_out(gb, b):
            row0 = base_row + gb * CHUNK
            pltpu.async_copy(obuf[b], out_hbm.at[pl.ds(row0, CHUNK)], osem[b])

        def wait_out(b):
            pltpu.make_async_copy(
                obuf[b], out_hbm.at[pl.ds(0, CHUNK)], osem[b]).wait()

        start_gather(0, 0)
        start_gather(1, 1)
        for b in (0, 1):
            wait_gather(b)
            scale(b)
            start_out(b, b)
            start_gather(b + 2, b)

        @pl.loop(2, nch - 2, step=2)
        def _(g):
            for b in (0, 1):
                gb = g + b
                wait_gather(b)
                wait_out(b)
                scale(b)
                start_out(gb, b)
                start_gather(gb + 2, b)

        for b in (0, 1):
            wait_gather(b)
            wait_out(b)
            scale(b)
            start_out(nch - 2 + b, b)
        for b in (0, 1):
            wait_out(b)

    return lookup


def kernel(x, table):
    rows, cols = x.shape
    n = rows * cols
    idx = x.reshape(n // CHUNK, CHUNK).astype(jnp.int32)
    out = _make_sc_lookup(n)(idx, table)
    return out.reshape(rows, cols, DIM)
